# Initial kernel scaffold; baseline (speedup 1.0000x reference)
import jax
import jax.numpy as jnp
from jax.experimental import pallas as pl


def kernel(x, edge_index, batch, params):
    raise NotImplementedError("write your pallas kernel here")


# speedup vs baseline: 2.5148x; 2.5148x over previous
import functools

import jax
import jax.numpy as jnp
from jax import lax
from jax.experimental import pallas as pl
from jax.experimental.pallas import tpu as pltpu
from jax.experimental.pallas import tpu_sc as plsc

N_PAD = 10240
BIG = 3.0e38
BIG_I = 2147483647


def _leaky(v):
    return jnp.where(v >= 0, v, 0.01 * v)


def _edge1_body(gd_ref, gs_ref, w1a_ref, w1b_ref, b1_ref, w2_ref, b2_ref,
                m_ref, *, e0, be):
    i = pl.program_id(0)
    gd = gd_ref[...]
    cat = jnp.concatenate([gd, gs_ref[...] - gd], axis=1)
    w1 = jnp.concatenate([w1a_ref[...], w1b_ref[...]], axis=0)
    z1 = jnp.dot(cat, w1, preferred_element_type=jnp.float32) + b1_ref[...]
    a = _leaky(z1)
    z = jnp.dot(a, w2_ref[...], preferred_element_type=jnp.float32) + b2_ref[...]
    m = _leaky(z)
    rows = i * be + lax.broadcasted_iota(jnp.int32, (be, 1), 0)
    m_ref[...] = jnp.where(rows < e0, m, 0.0)


def _edge1(gd, gs, w1, b1, w2, b2, e0):
    ep = gd.shape[0]
    be = 512
    return pl.pallas_call(
        functools.partial(_edge1_body, e0=e0, be=be),
        grid=(ep // be,),
        in_specs=[
            pl.BlockSpec((be, 128), lambda i: (i, 0)),
            pl.BlockSpec((be, 128), lambda i: (i, 0)),
            pl.BlockSpec((128, 128), lambda i: (0, 0)),
            pl.BlockSpec((128, 128), lambda i: (0, 0)),
            pl.BlockSpec((1, 128), lambda i: (0, 0)),
            pl.BlockSpec((128, 256), lambda i: (0, 0)),
            pl.BlockSpec((1, 256), lambda i: (0, 0)),
        ],
        out_specs=pl.BlockSpec((be, 256), lambda i: (i, 0)),
        out_shape=jax.ShapeDtypeStruct((ep, 256), jnp.float32),
    )(gd, gs, w1[:128], w1[128:], b1, w2, b2)


def _conv24_body(g_ref, h_ref, w1a_ref, w1b_ref, b1_ref, w2_ref, b2_ref, o_ref):
    nb, f = h_ref.shape
    hd = jnp.broadcast_to(h_ref[...][:, None, :], (nb, 8, f)).reshape(nb * 8, f)
    diff = g_ref[...] - hd
    z1 = (jnp.dot(hd, w1a_ref[...], preferred_element_type=jnp.float32)
          + jnp.dot(diff, w1b_ref[...], preferred_element_type=jnp.float32)
          ) + b1_ref[...]
    a = _leaky(z1)
    z = jnp.dot(a, w2_ref[...], preferred_element_type=jnp.float32) + b2_ref[...]
    m = _leaky(z).reshape(nb, 8, 256)
    acc = m[:, 0, :]
    for j in range(1, 8):
        acc = acc + m[:, j, :]
    o_ref[...] = acc


def _conv24(g, h, w1, b1, w2, b2):
    np_, f = h.shape
    hp = w1.shape[1]
    nb = 128
    return pl.pallas_call(
        _conv24_body,
        grid=(np_ // nb,),
        in_specs=[
            pl.BlockSpec((nb * 8, f), lambda i: (i, 0)),
            pl.BlockSpec((nb, f), lambda i: (i, 0)),
            pl.BlockSpec((f, hp), lambda i: (0, 0)),
            pl.BlockSpec((f, hp), lambda i: (0, 0)),
            pl.BlockSpec((1, hp), lambda i: (0, 0)),
            pl.BlockSpec((hp, 256), lambda i: (0, 0)),
            pl.BlockSpec((1, 256), lambda i: (0, 0)),
        ],
        out_specs=pl.BlockSpec((nb, 256), lambda i: (i, 0)),
        out_shape=jax.ShapeDtypeStruct((np_, 256), jnp.float32),
    )(g, h, w1[:f], w1[f:], b1, w2, b2)


def _knn_body(prow_ref, pt_ref, idx_ref, d2_ref, *, r, npad, n, k, ct):
    i = pl.program_id(0)
    nct = npad // ct
    p = prow_ref[...]
    sqr = jnp.sum(p * p, axis=1, keepdims=True)
    row_ids = i * r + lax.broadcasted_iota(jnp.int32, (r, 1), 0)

    def fill(t, _):
        pt = pt_ref[:, pl.ds(t * ct, ct)]
        sqc = jnp.sum(pt * pt, axis=0, keepdims=True)
        d = jnp.dot(p, pt, preferred_element_type=jnp.float32)
        d2 = sqr + sqc - 2.0 * d
        cols = t * ct + lax.broadcasted_iota(jnp.int32, (r, ct), 1)
        d2 = jnp.where((cols == row_ids) | (cols >= n), BIG, d2)
        d2_ref[:, pl.ds(t * ct, ct)] = d2
        return 0

    lax.fori_loop(0, nct, fill, 0)

    idx_acc = jnp.zeros((r, 128), jnp.int32)
    lanes = lax.broadcasted_iota(jnp.int32, (r, 128), 1)
    for kk in range(k):
        def pmin(t, m):
            v = d2_ref[:, pl.ds(t * ct, ct)]
            return jnp.minimum(m, jnp.min(v, axis=1, keepdims=True))

        m = lax.fori_loop(0, nct, pmin, jnp.full((r, 1), BIG, jnp.float32))

        def pidx(t, ix):
            v = d2_ref[:, pl.ds(t * ct, ct)]
            cols = t * ct + lax.broadcasted_iota(jnp.int32, (r, ct), 1)
            cand = jnp.where(v == m, cols, BIG_I)
            return jnp.minimum(ix, jnp.min(cand, axis=1, keepdims=True))

        ix = lax.fori_loop(0, nct, pidx, jnp.full((r, 1), BIG_I, jnp.int32))

        def pmask(t, _):
            v = d2_ref[:, pl.ds(t * ct, ct)]
            cols = t * ct + lax.broadcasted_iota(jnp.int32, (r, ct), 1)
            d2_ref[:, pl.ds(t * ct, ct)] = jnp.where(cols == ix, BIG, v)
            return 0

        lax.fori_loop(0, nct, pmask, 0)
        idx_acc = jnp.where(lanes == kk, ix, idx_acc)
    idx_ref[...] = idx_acc


def _knn(pos):
    npad = pos.shape[0]
    r = 256
    pt = jnp.transpose(pos[:, :8])
    return pl.pallas_call(
        functools.partial(_knn_body, r=r, npad=npad, n=10000, k=8, ct=512),
        grid=(npad // r,),
        in_specs=[
            pl.BlockSpec((r, 8), lambda i: (i, 0)),
            pl.BlockSpec((8, npad), lambda i: (0, 0)),
        ],
        out_specs=pl.BlockSpec((r, 128), lambda i: (i, 0)),
        out_shape=jax.ShapeDtypeStruct((npad, 128), jnp.int32),
        scratch_shapes=[pltpu.VMEM((r, npad), jnp.float32)],
    )(pos, pt)


def _head_body(h_ref, w1, b1, w2, b2, w3, b3, w4, b4, w5, b5, o_ref):
    h = h_ref[...]
    a = _leaky(jnp.dot(h, w1[...], preferred_element_type=jnp.float32) + b1[...])
    a = _leaky(jnp.dot(a, w2[...], preferred_element_type=jnp.float32) + b2[...])
    a = _leaky(jnp.dot(a, w3[...], preferred_element_type=jnp.float32) + b3[...])
    a = jnp.maximum(jnp.dot(a, w4[...], preferred_element_type=jnp.float32) + b4[...], 0.0)
    o_ref[...] = jnp.dot(a, w5[...], preferred_element_type=jnp.float32) + b5[...]


def _head(hcat, ws):
    np_, fin = hcat.shape
    r = 256
    specs = [pl.BlockSpec((r, fin), lambda i: (i, 0))]
    args = [hcat]
    for w, b in ws:
        specs.append(pl.BlockSpec(w.shape, lambda i: (0, 0)))
        specs.append(pl.BlockSpec(b.shape, lambda i: (0, 0)))
        args.extend([w, b])
    return pl.pallas_call(
        _head_body,
        grid=(np_ // r,),
        in_specs=specs,
        out_specs=pl.BlockSpec((r, 128), lambda i: (i, 0)),
        out_shape=jax.ShapeDtypeStruct((np_, 128), jnp.float32),
    )(*args)


def _sc_gather(table, idx):
    t, d = table.shape
    e = idx.shape[0]
    ch = 128
    per_w = e // 32
    nch = per_w // ch
    mesh = plsc.VectorSubcoreMesh(core_axis_name="c", subcore_axis_name="s")

    def body(table_hbm, idx_hbm, out_hbm, idx_v, rows_v, sem):
        wid = lax.axis_index("s") * 2 + lax.axis_index("c")
        base = wid * per_w

        def step(c, _):
            cb = base + c * ch
            pltpu.sync_copy(idx_hbm.at[pl.ds(cb, ch)], idx_v)
            pltpu.async_copy(table_hbm.at[idx_v], rows_v, sem).wait()
            pltpu.sync_copy(rows_v, out_hbm.at[pl.ds(cb, ch)])
            return 0

        lax.fori_loop(0, nch, step, 0)

    f = pl.kernel(
        body,
        out_type=jax.ShapeDtypeStruct((e, d), jnp.float32),
        mesh=mesh,
        scratch_types=[
            pltpu.VMEM((ch,), jnp.int32),
            pltpu.VMEM((ch, d), jnp.float32),
            pltpu.SemaphoreType.DMA,
        ],
    )
    return f(table, idx)


def _sc_scatter_sum(msg, dst, npad):
    e, d = msg.shape
    nw = 32
    nr = npad // nw
    dch = 4096
    ndch = e // dch
    mesh = plsc.VectorSubcoreMesh(core_axis_name="c", subcore_axis_name="s")

    ezero = e - 8

    def body(msg_hbm, dst_hbm, out_hbm, dst_v, ebuf_v, dbuf_v, rows_v, acc_v, sem):
        wid = lax.axis_index("s") * 2 + lax.axis_index("c")
        lo = wid * nr
        lane = lax.iota(jnp.int32, 16)

        def zstep(j, _):
            acc_v[pl.ds(j * 16, 16)] = jnp.zeros((16,), jnp.float32)
            return 0

        lax.fori_loop(0, nr * d // 16, zstep, 0)

        def chunk(cc, _):
            pltpu.sync_copy(dst_hbm.at[pl.ds(cc * dch, dch)], dst_v)

            def pstep(j, _):
                ebuf_v[pl.ds(j * 16, 16)] = jnp.full((16,), ezero, jnp.int32)
                dbuf_v[pl.ds(j * 16, 16)] = jnp.full((16,), lo, jnp.int32)
                return 0

            lax.fori_loop(0, (dch + 32) // 16, pstep, 0)

            dnums = lax.GatherDimensionNumbers(
                offset_dims=(), collapsed_slice_dims=(0,), start_index_map=(0,)
            )

            def _vgather(v, idxs):
                return lax.gather(
                    v, idxs[:, None], dnums, (1,),
                    mode=lax.GatherScatterMode.PROMISE_IN_BOUNDS,
                )

            def group(gi, cnt):
                dst16 = dst_v[pl.ds(gi * 16, 16)]
                msk = jnp.where(dst16 >= lo, 1, 0)
                msk = jnp.where(dst16 < lo + nr, msk, 0)
                csum = msk
                for sh in (1, 2, 4, 8):
                    shifted = _vgather(csum, jnp.maximum(lane - sh, 0))
                    csum = csum + jnp.where(lane >= sh, shifted, 0)
                cntg = csum[15]
                inv = jnp.zeros((16,), jnp.int32)
                for l in range(16):
                    inv = inv + jnp.where(lane + 1 == csum[l], msk[l] * l, 0)
                validk = lane < cntg
                ce = jnp.where(validk, cc * dch + gi * 16 + inv, ezero)
                cd = jnp.where(validk, _vgather(dst16, inv), lo)
                o = jnp.remainder(cnt, 16)
                rix = jnp.remainder(lane - o + 16, 16)
                rote = _vgather(ce, rix)
                rotd = _vgather(cd, rix)
                a = cnt - o
                up = lane >= o
                vA = ebuf_v[pl.ds(a, 16)]
                ebuf_v[pl.ds(a, 16)] = jnp.where(up, rote, vA)
                vB = ebuf_v[pl.ds(a + 16, 16)]
                ebuf_v[pl.ds(a + 16, 16)] = jnp.where(up, vB, rote)
                wA = dbuf_v[pl.ds(a, 16)]
                dbuf_v[pl.ds(a, 16)] = jnp.where(up, rotd, wA)
                wB = dbuf_v[pl.ds(a + 16, 16)]
                dbuf_v[pl.ds(a + 16, 16)] = jnp.where(up, wB, rotd)
                return cnt + cntg

            total = lax.fori_loop(0, dch // 16, group, jnp.int32(0))

            def batch(b, _):
                idx16 = ebuf_v[pl.ds(b * 16, 16)]
                dst16 = dbuf_v[pl.ds(b * 16, 16)]
                pltpu.async_copy(msg_hbm.at[idx16], rows_v, sem).wait()
                for l in range(16):
                    r = (dst16[l] - lo) * d
                    for j in range(d // 16):
                        plsc.addupdate(
                            acc_v.at[pl.ds(r + j * 16, 16)],
                            rows_v[l, pl.ds(j * 16, 16)],
                        )
                return 0

            lax.fori_loop(0, (total + 15) // 16, batch, 0)
            return 0

        lax.fori_loop(0, ndch, chunk, 0)
        pltpu.sync_copy(acc_v, out_hbm.at[pl.ds(lo * d, nr * d)])

    f = pl.kernel(
        body,
        out_type=jax.ShapeDtypeStruct((npad * d,), jnp.float32),
        mesh=mesh,
        scratch_types=[
            pltpu.VMEM((dch,), jnp.int32),
            pltpu.VMEM((dch + 32,), jnp.int32),
            pltpu.VMEM((dch + 32,), jnp.int32),
            pltpu.VMEM((16, d), jnp.float32),
            pltpu.VMEM((nr * d,), jnp.float32),
            pltpu.SemaphoreType.DMA,
        ],
    )
    return f(msg, dst).reshape(npad, d)


def _pad2(w, rows, cols):
    return jnp.zeros((rows, cols), jnp.float32).at[: w.shape[0], : w.shape[1]].set(w)


def kernel(x, edge_index, batch, params):
    n = x.shape[0]
    xp = jnp.zeros((N_PAD, 128), jnp.float32).at[:n].set(x)

    (w1, b1), (w2, b2) = params["convs"][0]
    e0 = edge_index.shape[1]
    ep = 163840
    srcp = jnp.zeros((ep,), jnp.int32).at[:e0].set(edge_index[0])
    dstp = jnp.zeros((ep,), jnp.int32).at[:e0].set(edge_index[1])
    gd = _sc_gather(xp, dstp)
    gs = _sc_gather(xp, srcp)
    msg = _edge1(gd, gs, w1, b1[None], w2, b2[None], e0)
    h = _sc_scatter_sum(msg, dstp, N_PAD)
    skips = [xp, h]

    for li in range(1, 4):
        (w1, b1), (w2, b2) = params["convs"][li]
        pos = jnp.zeros((N_PAD, 8), jnp.float32).at[:, :3].set(h[:, :3])
        idx = _knn(pos)[:, :8]
        src = idx.reshape(-1)
        g = _sc_gather(h, src)
        w1p = _pad2(w1, 512, 384)
        b1p = _pad2(b1[None], 1, 384)
        w2p = _pad2(w2, 384, 256)
        h = _conv24(g, h, w1p, b1p, w2p, b2[None])
        skips.append(h)

    hcat = jnp.concatenate(skips, axis=1)
    (pw1, pb1), (pw2, pb2) = params["post"]
    (rw, rb) = params["readout"][0]
    (hw1, hb1), (hw2, hb2) = params["head"]
    ws = [
        (_pad2(pw1, 1152, 384), _pad2(pb1[None], 1, 384)),
        (_pad2(pw2, 384, 256), pb2[None]),
        (rw, rb[None]),
        (_pad2(hw1, 128, 128), _pad2(hb1[None], 1, 128)),
        (_pad2(hw2, 128, 128), _pad2(hb2[None], 1, 128)),
    ]
    out = _head(hcat, ws)
    return out[:n, 0]
